# baseline (speedup 1.0000x reference)
import functools

import jax
import jax.numpy as jnp
from jax import lax
from jax.experimental import pallas as pl
from jax.experimental.pallas import tpu as pltpu
from jax.experimental.pallas import tpu_sc as plsc

_D = 129
_B = 16384
_NW = 32
_CHUNK = 128
_ROWS_PER_W = _B // _NW
_CHUNKS_PER_W = _ROWS_PER_W // _CHUNK


def _gather_body(idx_hbm, table_hbm, out_hbm, idx_v, rows_v, sem):
    wid = lax.axis_index("s") * 2 + lax.axis_index("c")
    base = wid * _CHUNKS_PER_W
    pltpu.sync_copy(idx_hbm.at[pl.ds(base, _CHUNKS_PER_W)], idx_v)
    copies = [
        pltpu.async_copy(table_hbm.at[idx_v.at[j]], rows_v.at[j], sem)
        for j in range(_CHUNKS_PER_W)
    ]
    for c in copies:
        c.wait()
    pltpu.sync_copy(rows_v, out_hbm.at[pl.ds(base, _CHUNKS_PER_W)])


@jax.jit
def _gather(idx2d, table):
    run = pl.kernel(
        _gather_body,
        out_type=jax.ShapeDtypeStruct((_B // _CHUNK, _CHUNK, _D), jnp.float32),
        mesh=plsc.VectorSubcoreMesh(core_axis_name="c", subcore_axis_name="s"),
        scratch_types=[
            pltpu.VMEM((_CHUNKS_PER_W, _CHUNK), jnp.int32),
            pltpu.VMEM((_CHUNKS_PER_W, _CHUNK, _D), jnp.float32),
            pltpu.SemaphoreType.DMA,
        ],
        compiler_params=pltpu.CompilerParams(use_tc_tiling_on_sc=False),
    )
    return run(idx2d, table)


def kernel(batch, table):
    idx2d = batch.astype(jnp.int32).reshape(_B // _CHUNK, _CHUNK)
    out = _gather(idx2d, table)
    return out.reshape(_B, _D)


# speedup vs baseline: 5.2900x; 5.2900x over previous
import functools

import jax
import jax.numpy as jnp
from jax import lax
from jax.experimental import pallas as pl
from jax.experimental.pallas import tpu as pltpu
from jax.experimental.pallas import tpu_sc as plsc

_D = 129
_DM = 128
_B = 16384
_NW = 32
_CHUNK = 128
_ROWS_PER_W = _B // _NW
_CPW = _ROWS_PER_W // _CHUNK


def _gather_body(idx_hbm, table_hbm, last_hbm, out_hbm, lastout_hbm,
                 idx_v, rows_v, last_v, sem):
    wid = lax.axis_index("s") * 2 + lax.axis_index("c")
    base = wid * _CPW
    pltpu.sync_copy(idx_hbm.at[pl.ds(base, _CPW)], idx_v)
    for j in range(_CPW):
        pltpu.async_copy(
            table_hbm.at[idx_v.at[j], pl.ds(0, _DM)], rows_v.at[j], sem
        ).wait()
        pltpu.async_copy(last_hbm.at[idx_v.at[j]], last_v.at[j], sem).wait()
    pltpu.sync_copy(rows_v, out_hbm.at[pl.ds(base, _CPW)])
    pltpu.sync_copy(last_v, lastout_hbm.at[pl.ds(base, _CPW)])


@jax.jit
def _gather(idx2d, table, tab_last):
    run = pl.kernel(
        _gather_body,
        out_type=(
            jax.ShapeDtypeStruct((_B // _CHUNK, _CHUNK, _DM), jnp.float32),
            jax.ShapeDtypeStruct((_B // _CHUNK, _CHUNK), jnp.float32),
        ),
        mesh=plsc.VectorSubcoreMesh(core_axis_name="c", subcore_axis_name="s"),
        scratch_types=[
            pltpu.VMEM((_CPW, _CHUNK), jnp.int32),
            pltpu.VMEM((_CPW, _CHUNK, _DM), jnp.float32),
            pltpu.VMEM((_CPW, _CHUNK), jnp.float32),
            pltpu.SemaphoreType.DMA,
        ],
    )
    return run(idx2d, table, tab_last)


def kernel(batch, table):
    idx2d = batch.astype(jnp.int32).reshape(_B // _CHUNK, _CHUNK)
    tab_last = table[:, _DM]
    main, last = _gather(idx2d, table, tab_last)
    return jnp.concatenate(
        [main.reshape(_B, _DM), last.reshape(_B, 1)], axis=1)


# speedup vs baseline: 5.3939x; 1.0196x over previous
import jax
import jax.numpy as jnp
from jax import lax
from jax.experimental import pallas as pl
from jax.experimental.pallas import tpu as pltpu
from jax.experimental.pallas import tpu_sc as plsc

_D = 129
_DM = 128
_B = 16384
_NW = 32
_CHUNK = 128
_ROWS_PER_W = _B // _NW
_CPW = _ROWS_PER_W // _CHUNK


def _gather_body(idx_hbm, main_hbm, last_hbm, out_hbm, lastout_hbm,
                 idx_v, rows_v, last_v, sem, sem2):
    wid = lax.axis_index("s") * 2 + lax.axis_index("c")
    base = wid * _CPW
    pltpu.sync_copy(idx_hbm.at[pl.ds(base, _CPW)], idx_v)
    copies = []
    for j in range(_CPW):
        copies.append(pltpu.async_copy(
            main_hbm.at[idx_v.at[j]], rows_v.at[j], sem))
        copies.append(pltpu.async_copy(
            last_hbm.at[idx_v.at[j]], last_v.at[j], sem2))
    for c in copies:
        c.wait()
    pltpu.sync_copy(rows_v, out_hbm.at[pl.ds(base, _CPW)])
    pltpu.sync_copy(last_v, lastout_hbm.at[pl.ds(base, _CPW)])


@jax.jit
def _gather(idx2d, tab_main, tab_last):
    run = pl.kernel(
        _gather_body,
        out_type=(
            jax.ShapeDtypeStruct((_B // _CHUNK, _CHUNK, _DM), jnp.float32),
            jax.ShapeDtypeStruct((_B // _CHUNK, _CHUNK), jnp.float32),
        ),
        mesh=plsc.VectorSubcoreMesh(core_axis_name="c", subcore_axis_name="s"),
        scratch_types=[
            pltpu.VMEM((_CPW, _CHUNK), jnp.int32),
            pltpu.VMEM((_CPW, _CHUNK, _DM), jnp.float32),
            pltpu.VMEM((_CPW, _CHUNK), jnp.float32),
            pltpu.SemaphoreType.DMA,
            pltpu.SemaphoreType.DMA,
        ],
    )
    return run(idx2d, tab_main, tab_last)


def kernel(batch, table):
    idx2d = batch.astype(jnp.int32).reshape(_B // _CHUNK, _CHUNK)
    tab_main = table[:, :_DM]
    tab_last = table[:, _DM]
    main, last = _gather(idx2d, tab_main, tab_last)
    return jnp.concatenate(
        [main.reshape(_B, _DM), last.reshape(_B, 1)], axis=1)


# speedup vs baseline: 6.9962x; 1.2970x over previous
import jax
import jax.numpy as jnp
from jax import lax
from jax.experimental import pallas as pl
from jax.experimental.pallas import tpu as pltpu
from jax.experimental.pallas import tpu_sc as plsc

_D = 129
_N = 100000
_B = 16384
_NW = 32
_RPW = _D // _NW
_L = 16
_OCHUNK = 4096


def _row_body(r, tabt_hbm, outt_hbm, idx_v, row_v, out_v, zeros16):
    pltpu.sync_copy(tabt_hbm.at[pl.ds(r, 1)], row_v)

    def gather16(b, cc):
        o = cc * _OCHUNK + b * _L
        idx16 = idx_v[0, pl.ds(o, _L)]
        out_v[0, pl.ds(b * _L, _L)] = plsc.load_gather(row_v, [zeros16, idx16])

    for cc in range(_B // _OCHUNK):
        lax.fori_loop(0, _OCHUNK // _L,
                      lambda b, c: (gather16(b, c), c)[1], cc)
        pltpu.sync_copy(out_v, outt_hbm.at[pl.ds(r, 1),
                                           pl.ds(cc * _OCHUNK, _OCHUNK)])


def _gather_body(idx_hbm, tabt_hbm, outt_hbm, idx_v, row_v, out_v):
    wid = lax.axis_index("s") * 2 + lax.axis_index("c")
    pltpu.sync_copy(idx_hbm, idx_v)
    zeros16 = jnp.zeros((_L,), jnp.int32)
    for t in range(_RPW):
        _row_body(wid * _RPW + t, tabt_hbm, outt_hbm,
                  idx_v, row_v, out_v, zeros16)
    @pl.when(wid == 0)
    def _():
        _row_body(_D - 1, tabt_hbm, outt_hbm, idx_v, row_v, out_v, zeros16)


def kernel(batch, table):
    idx2d = batch.astype(jnp.int32).reshape(1, _B)
    tabt = table.T
    run = pl.kernel(
        _gather_body,
        out_type=jax.ShapeDtypeStruct((_D, _B), jnp.float32),
        mesh=plsc.VectorSubcoreMesh(core_axis_name="c", subcore_axis_name="s"),
        scratch_types=[
            pltpu.VMEM((1, _B), jnp.int32),
            pltpu.VMEM((1, _N), jnp.float32),
            pltpu.VMEM((1, _OCHUNK), jnp.float32),
        ],
        compiler_params=pltpu.CompilerParams(needs_layout_passes=False),
    )
    outt = run(idx2d, tabt)
    return outt.T


# speedup vs baseline: 12.8574x; 1.8378x over previous
import jax
import jax.numpy as jnp
from jax import lax
from jax.experimental import pallas as pl
from jax.experimental.pallas import tpu as pltpu
from jax.experimental.pallas import tpu_sc as plsc

_D = 129
_N = 100000
_B = 16384
_NW = 32
_RPW = (_D - 1) // _NW
_L = 16
_OCHUNK = 4096
_SEG = _B // _NW


def _gather_body(idx_hbm, tabt_hbm, last_hbm, outt_hbm,
                 idx_v, row_v, ob0, ob1, last_v, sem_w, sem_g):
    wid = lax.axis_index("s") * 2 + lax.axis_index("c")
    pltpu.sync_copy(idx_hbm, idx_v)
    zeros16 = jnp.zeros((_L,), jnp.int32)

    seg = wid * _SEG
    last_copies = [
        pltpu.async_copy(
            last_hbm.at[idx_v.at[0, pl.ds(seg + j * 128, 128)]],
            last_v.at[0, pl.ds(j * 128, 128)], sem_g)
        for j in range(_SEG // 128)
    ]

    bufs = (ob0, ob1)
    pending = []

    def gather_chunk(buf, cbase):
        @plsc.parallel_loop(0, _OCHUNK // _L, unroll=8)
        def _(b):
            idx16 = idx_v[0, pl.ds(cbase + b * _L, _L)]
            buf[0, pl.ds(b * _L, _L)] = plsc.load_gather(
                row_v, [zeros16, idx16])

    for t in range(_RPW):
        r = wid * _RPW + t
        pltpu.sync_copy(tabt_hbm.at[pl.ds(r, 1)], row_v)
        for cc in range(_B // _OCHUNK):
            buf = bufs[cc % 2]
            if len(pending) >= 2:
                pending.pop(0).wait()
            gather_chunk(buf, cc * _OCHUNK)
            pending.append(pltpu.async_copy(
                buf, outt_hbm.at[pl.ds(r, 1), pl.ds(cc * _OCHUNK, _OCHUNK)],
                sem_w))
    while pending:
        pending.pop(0).wait()
    for c in last_copies:
        c.wait()
    pltpu.sync_copy(last_v, outt_hbm.at[pl.ds(_D - 1, 1), pl.ds(seg, _SEG)])


def kernel(batch, table):
    idx2d = batch.astype(jnp.int32).reshape(1, _B)
    tabt = table.T
    tab_last = table[:, _D - 1]
    run = pl.kernel(
        _gather_body,
        out_type=jax.ShapeDtypeStruct((_D, _B), jnp.float32),
        mesh=plsc.VectorSubcoreMesh(core_axis_name="c", subcore_axis_name="s"),
        scratch_types=[
            pltpu.VMEM((1, _B), jnp.int32),
            pltpu.VMEM((1, _N), jnp.float32),
            pltpu.VMEM((1, _OCHUNK), jnp.float32),
            pltpu.VMEM((1, _OCHUNK), jnp.float32),
            pltpu.VMEM((1, _SEG), jnp.float32),
            pltpu.SemaphoreType.DMA,
            pltpu.SemaphoreType.DMA,
        ],
        compiler_params=pltpu.CompilerParams(needs_layout_passes=False),
    )
    outt = run(idx2d, tabt, tab_last)
    return outt.T
